# Initial kernel scaffold; baseline (speedup 1.0000x reference)
import jax
import jax.numpy as jnp
from jax.experimental import pallas as pl


def kernel(x, gate_w, w13, w2):
    raise NotImplementedError("write your pallas kernel here")


# speedup vs baseline: 1.5083x; 1.5083x over previous
import functools

import jax
import jax.numpy as jnp
from jax.experimental import pallas as pl
from jax.experimental.pallas import tpu as pltpu

HIDDEN = 768
FFN = 1024
NUM_EXPERTS = 8
T = 2048
LANES = 128
NEG = -1e30


def _routing_body(x_ref, gw_ref, c_ref):
    x = x_ref[...]
    gw = gw_ref[...]
    logits = jax.lax.dot_general(
        x, gw, (((1,), (1,)), ((), ())), preferred_element_type=jnp.float32
    )
    lane = jax.lax.broadcasted_iota(jnp.int32, (T, LANES), 1)
    valid = lane < NUM_EXPERTS
    logits = jnp.where(valid, logits, NEG)
    m = jnp.max(logits, axis=1, keepdims=True)
    p = jnp.exp(logits - m)
    p = jnp.where(valid, p, 0.0)
    m1 = jnp.max(p, axis=1, keepdims=True)
    a1 = jnp.min(jnp.where(p == m1, lane, LANES), axis=1, keepdims=True)
    oh1 = (lane == a1).astype(jnp.float32)
    p2 = jnp.where(lane == a1, 0.0, p)
    m2 = jnp.max(p2, axis=1, keepdims=True)
    a2 = jnp.min(jnp.where(p2 == m2, lane, LANES), axis=1, keepdims=True)
    oh2 = (lane == a2).astype(jnp.float32)
    s = m1 + m2
    c_ref[...] = (m1 / s) * oh1 + (m2 / s) * oh2


def _moe_body(x_ref, c_ref, w13_ref, w2_ref, out_ref, *, bm):
    e = pl.program_id(0)
    i = pl.program_id(1)
    rows = pl.ds(i * bm, bm)
    xb = x_ref[rows, :]
    h = jax.lax.dot_general(
        xb, w13_ref[0], (((1,), (1,)), ((), ())),
        preferred_element_type=jnp.float32,
    )
    h1 = h[:, :FFN]
    h3 = h[:, FFN:]
    inter = h1 * (1.0 / (1.0 + jnp.exp(-h1))) * h3
    y = jax.lax.dot_general(
        inter, w2_ref[0], (((1,), (1,)), ((), ())),
        preferred_element_type=jnp.float32,
    )
    lane = jax.lax.broadcasted_iota(jnp.int32, (bm, LANES), 1)
    cb = jnp.sum(c_ref[rows, :] * (lane == e).astype(jnp.float32), axis=1,
                 keepdims=True)
    contrib = cb * y

    @pl.when(e == 0)
    def _():
        out_ref[rows, :] = contrib

    @pl.when(e > 0)
    def _():
        out_ref[rows, :] = out_ref[rows, :] + contrib


def kernel(x, gate_w, w13, w2):
    gw_pad = jnp.zeros((LANES, HIDDEN), jnp.float32).at[:NUM_EXPERTS].set(gate_w)
    c = pl.pallas_call(
        _routing_body,
        out_shape=jax.ShapeDtypeStruct((T, LANES), jnp.float32),
    )(x, gw_pad)

    bm = 256
    nt = T // bm
    out = pl.pallas_call(
        functools.partial(_moe_body, bm=bm),
        grid=(NUM_EXPERTS, nt),
        in_specs=[
            pl.BlockSpec((T, HIDDEN), lambda e, i: (0, 0)),
            pl.BlockSpec((T, LANES), lambda e, i: (0, 0)),
            pl.BlockSpec((1, 2 * FFN, HIDDEN), lambda e, i: (e, 0, 0)),
            pl.BlockSpec((1, HIDDEN, FFN), lambda e, i: (e, 0, 0)),
        ],
        out_specs=pl.BlockSpec((T, HIDDEN), lambda e, i: (0, 0)),
        out_shape=jax.ShapeDtypeStruct((T, HIDDEN), jnp.float32),
    )(x, c, w13, w2)
    return out
